# baseline (speedup 1.0000x reference)
import functools

import jax
import jax.numpy as jnp
from jax import lax
from jax.experimental import pallas as pl
from jax.experimental.pallas import tpu as pltpu
from jax.experimental.pallas import tpu_sc as plsc

_N_DST = [16384, 4096, 1024]
_N_OUT = 1024
_NH = 512

_BD = [128, 64, 32]
_CH = [128, 80, 96]
_NW = 32


def _seg_sum_sc(d, n_dst, B_d, C, E):
    NB = n_dst // B_d
    NBPW = NB // _NW
    JP = max(d // 256, 1)
    JV = d // (16 * JP)
    mesh = plsc.VectorSubcoreMesh(core_axis_name="c", subcore_axis_name="s")

    @functools.partial(
        pl.kernel,
        mesh=mesh,
        out_type=jax.ShapeDtypeStruct((n_dst, d), jnp.float32),
        scratch_types=[
            pltpu.VMEM((B_d, d), jnp.float32),
            pltpu.VMEM((C, d), jnp.float32),
            pltpu.VMEM((C, d), jnp.float32),
            pltpu.VMEM((C,), jnp.int32),
            pltpu.VMEM((C,), jnp.int32),
            pltpu.VMEM((B_d + 16,), jnp.int32),
            pltpu.VMEM((B_d + 16,), jnp.int32),
            pltpu.SemaphoreType.DMA,
            pltpu.SemaphoreType.DMA,
        ],
    )
    def k(h_hbm, src_hbm, olo_hbm, ohi_hbm, sum_hbm,
          acc, rbA, rbB, sxA, sxB, olo, ohi, smA, smB):
        cid = lax.axis_index("c")
        sid = lax.axis_index("s")
        wid = sid * 2 + cid
        zero16 = jnp.zeros((16,), jnp.float32)
        iota16 = lax.iota(jnp.int32, 16)

        def issue(base, sidx, rbuf, sem):
            pltpu.sync_copy(src_hbm.at[pl.ds(base, C)], sidx)
            pltpu.async_copy(h_hbm.at[sidx], rbuf, sem)

        def wait(sidx, rbuf, sem):
            pltpu.make_async_copy(h_hbm.at[sidx], rbuf, sem).wait()

        def process(rbuf, base):
            one16 = jnp.ones((16,), jnp.int32)
            zro16 = jnp.zeros((16,), jnp.int32)
            r0v = zro16
            r1v = zro16
            for g in range(B_d // 16):
                lo_g = olo[pl.ds(g * 16, 16)]
                hi_g = ohi[pl.ds(g * 16, 16)]
                r0v = r0v + jnp.where(hi_g <= base, one16, zro16)
                r1v = r1v + jnp.where(lo_g < base + C, one16, zro16)
            for sh in (8, 4, 2, 1):
                perm = jnp.bitwise_xor(iota16, sh)
                r0v = r0v + r0v.at[perm].get(mode="promise_in_bounds")
                r1v = r1v + r1v.at[perm].get(mode="promise_in_bounds")

            def rb(r, _):
                lo_r = olo[pl.ds(r, 16)][0]
                hi_r = ohi[pl.ds(r, 16)][0]
                es = jnp.maximum(lo_r - base, 0)
                ee = jnp.minimum(hi_r - base, C)

                @pl.when(ee > es)
                def _():
                    eee = jnp.maximum(ee, es)
                    for jh in range(JP):
                        o = jh * JV * 16

                        def eb(e, carry):
                            return tuple(
                                carry[j] + rbuf[e, pl.ds(o + j * 16, 16)]
                                for j in range(JV))

                        init = tuple(
                            acc[r, pl.ds(o + j * 16, 16)] for j in range(JV))
                        res = lax.fori_loop(es, eee, eb, init)
                        for j in range(JV):
                            acc[r, pl.ds(o + j * 16, 16)] = res[j]

                return 0

            lax.fori_loop(r0v[0], r1v[0], rb, 0)

        for kk in range(NBPW):
            b = wid * NBPW + kk
            lo = b * B_d
            pltpu.sync_copy(olo_hbm.at[pl.ds(lo, B_d)], olo.at[pl.ds(0, B_d)])
            pltpu.sync_copy(ohi_hbm.at[pl.ds(lo, B_d)], ohi.at[pl.ds(0, B_d)])

            def zrow(r, _):
                for j in range(d // 16):
                    acc[r, pl.ds(j * 16, 16)] = zero16
                return 0

            lax.fori_loop(0, B_d, zrow, 0)

            s0 = olo[pl.ds(0, 16)][0]
            e_end = ohi[pl.ds(B_d - 16, 16)][15]
            s0a = (s0 // 8) * 8
            nch = (e_end - s0a + C - 1) // C

            @pl.when(nch > 0)
            def _():
                issue(s0a, sxA, rbA, smA)

            def chunk(ct, _):
                b0 = s0a + ct * C

                @pl.when(ct % 2 == 0)
                def _():
                    @pl.when(ct + 1 < nch)
                    def _():
                        issue(b0 + C, sxB, rbB, smB)

                    wait(sxA, rbA, smA)
                    process(rbA, b0)

                @pl.when(ct % 2 == 1)
                def _():
                    @pl.when(ct + 1 < nch)
                    def _():
                        issue(b0 + C, sxA, rbA, smA)

                    wait(sxB, rbB, smB)
                    process(rbB, b0)

                return 0

            lax.fori_loop(0, nch, chunk, 0)
            pltpu.sync_copy(acc, sum_hbm.at[pl.ds(lo, B_d)])

    return k


def _dense_fused(n_dst, d_in, RB, d_res, with_proj, n_h):
    NBLK = n_dst // RB
    n_cb = _N_OUT // RB
    inv_n = 1.0 / float(n_dst)

    def body(h_ref, s_ref, ol_ref, oh_ref, ws_ref, wn_ref, g_ref, be_ref,
             r_ref, wr_ref, br_ref, o_ref, co_ref, y_scr, st_scr):
        p = pl.program_id(0)
        i = pl.program_id(1)

        @pl.when(p == 0)
        def _():
            cnt = (oh_ref[...] - ol_ref[...]).astype(jnp.float32)
            hn = s_ref[...] / jnp.maximum(cnt, 1.0)
            y = (jnp.dot(h_ref[...], ws_ref[...],
                         preferred_element_type=jnp.float32)
                 + jnp.dot(hn, wn_ref[...],
                           preferred_element_type=jnp.float32))
            y_scr[pl.ds(i * RB, RB), :] = y

            @pl.when(i == 0)
            def _():
                st_scr[...] = jnp.zeros_like(st_scr)

            st_scr[0:1, :] += jnp.sum(y, axis=0, keepdims=True)
            st_scr[1:2, :] += jnp.sum(y * y, axis=0, keepdims=True)

        @pl.when(p == 1)
        def _():
            mu = st_scr[0:1, :] * inv_n
            var = st_scr[1:2, :] * inv_n - mu * mu
            scale = g_ref[...] * lax.rsqrt(var + 1e-5)
            y = y_scr[pl.ds(i * RB, RB), :]
            hb = jnp.maximum((y - mu) * scale + be_ref[...], 0.0)

            @pl.when(i < n_cb)
            def _():
                co_ref[...] = hb

            if with_proj:
                res = (jnp.dot(r_ref[...], wr_ref[...],
                               preferred_element_type=jnp.float32)
                       + br_ref[...])
            else:
                res = r_ref[...]
            o_ref[...] = hb + res

    ph0 = lambda p, i: (jnp.where(p == 0, i, 0), 0)
    ph1 = lambda p, i: (jnp.where(p == 1, i, 0), 0)
    fix = lambda p, i: (0, 0)
    return pl.pallas_call(
        body,
        grid=(2, NBLK),
        in_specs=[
            pl.BlockSpec((RB, d_in), ph0),
            pl.BlockSpec((RB, d_in), ph0),
            pl.BlockSpec((RB, 1), ph0),
            pl.BlockSpec((RB, 1), ph0),
            pl.BlockSpec((d_in, _NH), fix),
            pl.BlockSpec((d_in, _NH), fix),
            pl.BlockSpec((1, _NH), fix),
            pl.BlockSpec((1, _NH), fix),
            pl.BlockSpec((RB, d_res), ph1),
            pl.BlockSpec((d_res, _NH), fix),
            pl.BlockSpec((1, _NH), fix),
        ],
        out_specs=[
            pl.BlockSpec((RB, _NH), ph1),
            pl.BlockSpec((RB, _NH),
                         lambda p, i: (jnp.where(p == 1, jnp.minimum(i, n_cb - 1), 0), 0)),
        ],
        out_shape=[
            jax.ShapeDtypeStruct((n_dst, _NH), jnp.float32),
            jax.ShapeDtypeStruct((_N_OUT, _NH), jnp.float32),
        ],
        scratch_shapes=[
            pltpu.VMEM((n_dst, _NH), jnp.float32),
            pltpu.VMEM((8, _NH), jnp.float32),
        ],
    )


def _mlp(d0, n_cls):
    d_cat = d0 + 3 * _NH

    def body(xp_ref, c1_ref, c2_ref, c3_ref, w1_ref, b1_ref, w2_ref, b2_ref,
             o_ref):
        h = (jnp.dot(xp_ref[...], w1_ref[0:d0, :],
                     preferred_element_type=jnp.float32)
             + jnp.dot(c1_ref[...], w1_ref[d0:d0 + _NH, :],
                       preferred_element_type=jnp.float32)
             + jnp.dot(c2_ref[...], w1_ref[d0 + _NH:d0 + 2 * _NH, :],
                       preferred_element_type=jnp.float32)
             + jnp.dot(c3_ref[...], w1_ref[d0 + 2 * _NH:d_cat, :],
                       preferred_element_type=jnp.float32)
             + b1_ref[...])
        o_ref[...] = (jnp.dot(h, w2_ref[...],
                              preferred_element_type=jnp.float32) + b2_ref[...])

    return pl.pallas_call(
        body,
        grid=(1,),
        in_specs=[
            pl.BlockSpec((_N_OUT, d0), lambda i: (0, 0)),
            pl.BlockSpec((_N_OUT, _NH), lambda i: (0, 0)),
            pl.BlockSpec((_N_OUT, _NH), lambda i: (0, 0)),
            pl.BlockSpec((_N_OUT, _NH), lambda i: (0, 0)),
            pl.BlockSpec((d_cat, 2 * n_cls), lambda i: (0, 0)),
            pl.BlockSpec((1, 2 * n_cls), lambda i: (0, 0)),
            pl.BlockSpec((2 * n_cls, n_cls), lambda i: (0, 0)),
            pl.BlockSpec((1, n_cls), lambda i: (0, 0)),
        ],
        out_specs=pl.BlockSpec((_N_OUT, n_cls), lambda i: (0, 0)),
        out_shape=jax.ShapeDtypeStruct((_N_OUT, n_cls), jnp.float32),
    )


def kernel(x, edge_src_0, edge_dst_0, edge_src_1, edge_dst_1, edge_src_2,
           edge_dst_2, W_self_0, W_neigh_0, gamma_0, beta_0, W_self_1,
           W_neigh_1, gamma_1, beta_1, W_self_2, W_neigh_2, gamma_2, beta_2,
           W_res, b_res, W_mlp1, b_mlp1, W_mlp2, b_mlp2):
    srcs = [edge_src_0, edge_src_1, edge_src_2]
    dsts = [edge_dst_0, edge_dst_1, edge_dst_2]
    Ws = [W_self_0, W_self_1, W_self_2]
    Wn = [W_neigh_0, W_neigh_1, W_neigh_2]
    gs = [gamma_0.reshape(1, -1), gamma_1.reshape(1, -1), gamma_2.reshape(1, -1)]
    bs = [beta_0.reshape(1, -1), beta_1.reshape(1, -1), beta_2.reshape(1, -1)]

    h = x
    colls = [x[:_N_OUT]]
    for l in range(3):
        n_dst = _N_DST[l]
        d_in = h.shape[1]
        B_d = _BD[l]
        src, dst = srcs[l], dsts[l]
        E = src.shape[0]
        C = _CH[l]

        hist = jnp.zeros((n_dst,), jnp.int32).at[dst].add(1, mode="drop")
        off_hi = jnp.cumsum(hist, dtype=jnp.int32)
        off_lo = off_hi - hist
        src_p = jnp.concatenate([src, jnp.zeros((2 * C,), jnp.int32)])

        sums = _seg_sum_sc(d_in, n_dst, B_d, C, E)(h, src_p, off_lo, off_hi)

        RB = 512
        if l == 0:
            h, coll = _dense_fused(n_dst, d_in, RB, d_in, True, h.shape[0])(
                h, sums, off_lo.reshape(n_dst, 1),
                off_hi.reshape(n_dst, 1), Ws[l], Wn[l], gs[l], bs[l],
                h, W_res, b_res.reshape(1, -1))
        else:
            h, coll = _dense_fused(n_dst, d_in, RB, _NH, False, h.shape[0])(
                h, sums, off_lo.reshape(n_dst, 1),
                off_hi.reshape(n_dst, 1), Ws[l], Wn[l], gs[l], bs[l],
                h, jnp.zeros((_NH, _NH), jnp.float32),
                jnp.zeros((1, _NH), jnp.float32))
        colls.append(coll)

    out = _mlp(x.shape[1], W_mlp2.shape[1])(
        colls[0], colls[1], colls[2], colls[3], W_mlp1,
        b_mlp1.reshape(1, -1), W_mlp2, b_mlp2.reshape(1, -1))
    return out


# speedup vs baseline: 1.0363x; 1.0363x over previous
import functools

import jax
import jax.numpy as jnp
from jax import lax
from jax.experimental import pallas as pl
from jax.experimental.pallas import tpu as pltpu
from jax.experimental.pallas import tpu_sc as plsc

_N_DST = [16384, 4096, 1024]
_N_OUT = 1024
_NH = 512

_BD = [128, 64, 32]
_CH = [128, 80, 96]
_NW = 32


def _seg_sum_sc(d, n_dst, B_d, C, E):
    NB = n_dst // B_d
    NBPW = NB // _NW
    JP = max(d // 256, 1)
    WP = d // 2 // JP
    mesh = plsc.VectorSubcoreMesh(core_axis_name="c", subcore_axis_name="s")

    @functools.partial(
        pl.kernel,
        mesh=mesh,
        out_type=jax.ShapeDtypeStruct((n_dst, d), jnp.float32),
        scratch_types=[
            pltpu.VMEM((B_d, d), jnp.float32),
            pltpu.VMEM((C, d // 2), jnp.int32),
            pltpu.VMEM((C, d // 2), jnp.int32),
            pltpu.VMEM((C,), jnp.int32),
            pltpu.VMEM((C,), jnp.int32),
            pltpu.VMEM((B_d + 16,), jnp.int32),
            pltpu.VMEM((B_d + 16,), jnp.int32),
            pltpu.SemaphoreType.DMA,
            pltpu.SemaphoreType.DMA,
        ],
    )
    def k(h_hbm, src_hbm, olo_hbm, ohi_hbm, sum_hbm,
          acc, rbA, rbB, sxA, sxB, olo, ohi, smA, smB):
        cid = lax.axis_index("c")
        sid = lax.axis_index("s")
        wid = sid * 2 + cid
        zero16 = jnp.zeros((16,), jnp.float32)
        iota16 = lax.iota(jnp.int32, 16)

        def issue(base, sidx, rbuf, sem):
            pltpu.sync_copy(src_hbm.at[pl.ds(base, C)], sidx)
            pltpu.async_copy(h_hbm.at[sidx], rbuf, sem)

        def wait(sidx, rbuf, sem):
            pltpu.make_async_copy(h_hbm.at[sidx], rbuf, sem).wait()

        def process(rbuf, base):
            one16 = jnp.ones((16,), jnp.int32)
            zro16 = jnp.zeros((16,), jnp.int32)
            r0v = zro16
            r1v = zro16
            for g in range(B_d // 16):
                lo_g = olo[pl.ds(g * 16, 16)]
                hi_g = ohi[pl.ds(g * 16, 16)]
                r0v = r0v + jnp.where(hi_g <= base, one16, zro16)
                r1v = r1v + jnp.where(lo_g < base + C, one16, zro16)
            for sh in (8, 4, 2, 1):
                perm = jnp.bitwise_xor(iota16, sh)
                r0v = r0v + r0v.at[perm].get(mode="promise_in_bounds")
                r1v = r1v + r1v.at[perm].get(mode="promise_in_bounds")

            def rb(r, _):
                lo_r = olo[pl.ds(r, 16)][0]
                hi_r = ohi[pl.ds(r, 16)][0]
                es = jnp.maximum(lo_r - base, 0)
                ee = jnp.minimum(hi_r - base, C)

                @pl.when(ee > es)
                def _():
                    eee = jnp.maximum(ee, es)
                    for jh in range(JP):
                        wo = jh * WP
                        cols = [(wo + j2 * 16, d // 2 + wo + j2 * 16)
                                for j2 in range(WP // 16)]

                        def eb(e, carry):
                            out = []
                            for j2 in range(WP // 16):
                                w = rbuf[e, pl.ds(wo + j2 * 16, 16)]
                                lo = lax.bitcast_convert_type(
                                    jnp.left_shift(w, 16), jnp.float32)
                                hi = lax.bitcast_convert_type(
                                    w & jnp.int32(-65536), jnp.float32)
                                out.append(carry[2 * j2] + lo)
                                out.append(carry[2 * j2 + 1] + hi)
                            return tuple(out)

                        init = sum(
                            ((acc[r, pl.ds(ca, 16)], acc[r, pl.ds(cb, 16)])
                             for ca, cb in cols), ())
                        res = lax.fori_loop(es, eee, eb, init)
                        for j2, (ca, cb) in enumerate(cols):
                            acc[r, pl.ds(ca, 16)] = res[2 * j2]
                            acc[r, pl.ds(cb, 16)] = res[2 * j2 + 1]

                return 0

            lax.fori_loop(r0v[0], r1v[0], rb, 0)

        for kk in range(NBPW):
            b = wid * NBPW + kk
            lo = b * B_d
            pltpu.sync_copy(olo_hbm.at[pl.ds(lo, B_d)], olo.at[pl.ds(0, B_d)])
            pltpu.sync_copy(ohi_hbm.at[pl.ds(lo, B_d)], ohi.at[pl.ds(0, B_d)])

            def zrow(r, _):
                for j in range(d // 16):
                    acc[r, pl.ds(j * 16, 16)] = zero16
                return 0

            lax.fori_loop(0, B_d, zrow, 0)

            s0 = olo[pl.ds(0, 16)][0]
            e_end = ohi[pl.ds(B_d - 16, 16)][15]
            s0a = (s0 // 8) * 8
            nch = (e_end - s0a + C - 1) // C

            @pl.when(nch > 0)
            def _():
                issue(s0a, sxA, rbA, smA)

            def chunk(ct, _):
                b0 = s0a + ct * C

                @pl.when(ct % 2 == 0)
                def _():
                    @pl.when(ct + 1 < nch)
                    def _():
                        issue(b0 + C, sxB, rbB, smB)

                    wait(sxA, rbA, smA)
                    process(rbA, b0)

                @pl.when(ct % 2 == 1)
                def _():
                    @pl.when(ct + 1 < nch)
                    def _():
                        issue(b0 + C, sxA, rbA, smA)

                    wait(sxB, rbB, smB)
                    process(rbB, b0)

                return 0

            lax.fori_loop(0, nch, chunk, 0)
            pltpu.sync_copy(acc, sum_hbm.at[pl.ds(lo, B_d)])

    return k


def _dense_fused(n_dst, d_in, RB, d_res, with_proj, n_h):
    NBLK = n_dst // RB
    n_cb = _N_OUT // RB
    inv_n = 1.0 / float(n_dst)

    def body(h_ref, s_ref, ol_ref, oh_ref, ws_ref, wn_ref, g_ref, be_ref,
             r_ref, wr_ref, br_ref, o_ref, co_ref, ob_ref, y_scr, st_scr):
        p = pl.program_id(0)
        i = pl.program_id(1)

        @pl.when(p == 0)
        def _():
            cnt = (oh_ref[...] - ol_ref[...]).astype(jnp.float32)
            hn = s_ref[...] / jnp.maximum(cnt, 1.0)
            y = (jnp.dot(h_ref[...], ws_ref[...],
                         preferred_element_type=jnp.float32)
                 + jnp.dot(hn, wn_ref[...],
                           preferred_element_type=jnp.float32))
            y_scr[pl.ds(i * RB, RB), :] = y

            @pl.when(i == 0)
            def _():
                st_scr[...] = jnp.zeros_like(st_scr)

            st_scr[0:1, :] += jnp.sum(y, axis=0, keepdims=True)
            st_scr[1:2, :] += jnp.sum(y * y, axis=0, keepdims=True)

        @pl.when(p == 1)
        def _():
            mu = st_scr[0:1, :] * inv_n
            var = st_scr[1:2, :] * inv_n - mu * mu
            scale = g_ref[...] * lax.rsqrt(var + 1e-5)
            y = y_scr[pl.ds(i * RB, RB), :]
            hb = jnp.maximum((y - mu) * scale + be_ref[...], 0.0)

            @pl.when(i < n_cb)
            def _():
                co_ref[...] = hb

            if with_proj:
                res = (jnp.dot(r_ref[...], wr_ref[...],
                               preferred_element_type=jnp.float32)
                       + br_ref[...])
            else:
                res = r_ref[...]
            out = hb + res
            o_ref[...] = out
            a16 = out[:, :_NH // 2].astype(jnp.bfloat16)
            b16 = out[:, _NH // 2:].astype(jnp.bfloat16)
            aw = lax.bitcast_convert_type(a16, jnp.uint16).astype(jnp.int32)
            bw = lax.bitcast_convert_type(b16, jnp.uint16).astype(jnp.int32)
            ob_ref[...] = aw | (bw << 16)

    ph0 = lambda p, i: (jnp.where(p == 0, i, 0), 0)
    ph1 = lambda p, i: (jnp.where(p == 1, i, 0), 0)
    fix = lambda p, i: (0, 0)
    return pl.pallas_call(
        body,
        grid=(2, NBLK),
        in_specs=[
            pl.BlockSpec((RB, d_in), ph0),
            pl.BlockSpec((RB, d_in), ph0),
            pl.BlockSpec((RB, 1), ph0),
            pl.BlockSpec((RB, 1), ph0),
            pl.BlockSpec((d_in, _NH), fix),
            pl.BlockSpec((d_in, _NH), fix),
            pl.BlockSpec((1, _NH), fix),
            pl.BlockSpec((1, _NH), fix),
            pl.BlockSpec((RB, d_res), ph1),
            pl.BlockSpec((d_res, _NH), fix),
            pl.BlockSpec((1, _NH), fix),
        ],
        out_specs=[
            pl.BlockSpec((RB, _NH), ph1),
            pl.BlockSpec((RB, _NH),
                         lambda p, i: (jnp.where(p == 1, jnp.minimum(i, n_cb - 1), 0), 0)),
            pl.BlockSpec((RB, _NH // 2), ph1),
        ],
        out_shape=[
            jax.ShapeDtypeStruct((n_dst, _NH), jnp.float32),
            jax.ShapeDtypeStruct((_N_OUT, _NH), jnp.float32),
            jax.ShapeDtypeStruct((n_dst, _NH // 2), jnp.int32),
        ],
        scratch_shapes=[
            pltpu.VMEM((n_dst, _NH), jnp.float32),
            pltpu.VMEM((8, _NH), jnp.float32),
        ],
    )


def _mlp(d0, n_cls):
    d_cat = d0 + 3 * _NH

    def body(xp_ref, c1_ref, c2_ref, c3_ref, w1_ref, b1_ref, w2_ref, b2_ref,
             o_ref):
        h = (jnp.dot(xp_ref[...], w1_ref[0:d0, :],
                     preferred_element_type=jnp.float32)
             + jnp.dot(c1_ref[...], w1_ref[d0:d0 + _NH, :],
                       preferred_element_type=jnp.float32)
             + jnp.dot(c2_ref[...], w1_ref[d0 + _NH:d0 + 2 * _NH, :],
                       preferred_element_type=jnp.float32)
             + jnp.dot(c3_ref[...], w1_ref[d0 + 2 * _NH:d_cat, :],
                       preferred_element_type=jnp.float32)
             + b1_ref[...])
        o_ref[...] = (jnp.dot(h, w2_ref[...],
                              preferred_element_type=jnp.float32) + b2_ref[...])

    return pl.pallas_call(
        body,
        grid=(1,),
        in_specs=[
            pl.BlockSpec((_N_OUT, d0), lambda i: (0, 0)),
            pl.BlockSpec((_N_OUT, _NH), lambda i: (0, 0)),
            pl.BlockSpec((_N_OUT, _NH), lambda i: (0, 0)),
            pl.BlockSpec((_N_OUT, _NH), lambda i: (0, 0)),
            pl.BlockSpec((d_cat, 2 * n_cls), lambda i: (0, 0)),
            pl.BlockSpec((1, 2 * n_cls), lambda i: (0, 0)),
            pl.BlockSpec((2 * n_cls, n_cls), lambda i: (0, 0)),
            pl.BlockSpec((1, n_cls), lambda i: (0, 0)),
        ],
        out_specs=pl.BlockSpec((_N_OUT, n_cls), lambda i: (0, 0)),
        out_shape=jax.ShapeDtypeStruct((_N_OUT, n_cls), jnp.float32),
    )


def kernel(x, edge_src_0, edge_dst_0, edge_src_1, edge_dst_1, edge_src_2,
           edge_dst_2, W_self_0, W_neigh_0, gamma_0, beta_0, W_self_1,
           W_neigh_1, gamma_1, beta_1, W_self_2, W_neigh_2, gamma_2, beta_2,
           W_res, b_res, W_mlp1, b_mlp1, W_mlp2, b_mlp2):
    srcs = [edge_src_0, edge_src_1, edge_src_2]
    dsts = [edge_dst_0, edge_dst_1, edge_dst_2]
    Ws = [W_self_0, W_self_1, W_self_2]
    Wn = [W_neigh_0, W_neigh_1, W_neigh_2]
    gs = [gamma_0.reshape(1, -1), gamma_1.reshape(1, -1), gamma_2.reshape(1, -1)]
    bs = [beta_0.reshape(1, -1), beta_1.reshape(1, -1), beta_2.reshape(1, -1)]

    def _pack(hf):
        d2 = hf.shape[1] // 2
        h16 = hf.astype(jnp.bfloat16)
        aw = lax.bitcast_convert_type(h16[:, :d2], jnp.uint16).astype(jnp.int32)
        bw = lax.bitcast_convert_type(h16[:, d2:], jnp.uint16).astype(jnp.int32)
        return aw | (bw << 16)

    h = x
    h_bf = _pack(x)
    colls = [x[:_N_OUT]]
    for l in range(3):
        n_dst = _N_DST[l]
        d_in = h.shape[1]
        B_d = _BD[l]
        src, dst = srcs[l], dsts[l]
        E = src.shape[0]
        C = _CH[l]

        hist = jnp.zeros((n_dst,), jnp.int32).at[dst].add(1, mode="drop")
        off_hi = jnp.cumsum(hist, dtype=jnp.int32)
        off_lo = off_hi - hist
        src_p = jnp.concatenate([src, jnp.zeros((2 * C,), jnp.int32)])

        sums = _seg_sum_sc(d_in, n_dst, B_d, C, E)(h_bf, src_p, off_lo, off_hi)

        RB = 512
        if l == 0:
            h, coll, h_bf = _dense_fused(n_dst, d_in, RB, d_in, True, h.shape[0])(
                h, sums, off_lo.reshape(n_dst, 1),
                off_hi.reshape(n_dst, 1), Ws[l], Wn[l], gs[l], bs[l],
                h, W_res, b_res.reshape(1, -1))
        else:
            h, coll, h_bf = _dense_fused(n_dst, d_in, RB, _NH, False, h.shape[0])(
                h, sums, off_lo.reshape(n_dst, 1),
                off_hi.reshape(n_dst, 1), Ws[l], Wn[l], gs[l], bs[l],
                h, jnp.zeros((_NH, _NH), jnp.float32),
                jnp.zeros((1, _NH), jnp.float32))
        colls.append(coll)

    out = _mlp(x.shape[1], W_mlp2.shape[1])(
        colls[0], colls[1], colls[2], colls[3], W_mlp1,
        b_mlp1.reshape(1, -1), W_mlp2, b_mlp2.reshape(1, -1))
    return out


# speedup vs baseline: 1.0522x; 1.0154x over previous
import functools

import jax
import jax.numpy as jnp
from jax import lax
from jax.experimental import pallas as pl
from jax.experimental.pallas import tpu as pltpu
from jax.experimental.pallas import tpu_sc as plsc

_N_DST = [16384, 4096, 1024]
_N_OUT = 1024
_NH = 512

_BD = [256, 128, 32]
_CH = [128, 80, 96]
_NW = 32


def _seg_sum_sc(d, n_dst, B_d, C, E):
    NB = n_dst // B_d
    NBPW = NB // _NW
    JP = max(d // 256, 1)
    WP = d // 2 // JP
    mesh = plsc.VectorSubcoreMesh(core_axis_name="c", subcore_axis_name="s")

    @functools.partial(
        pl.kernel,
        mesh=mesh,
        out_type=jax.ShapeDtypeStruct((n_dst, d), jnp.float32),
        scratch_types=[
            pltpu.VMEM((B_d, d), jnp.float32),
            pltpu.VMEM((C, d // 2), jnp.int32),
            pltpu.VMEM((C, d // 2), jnp.int32),
            pltpu.VMEM((C,), jnp.int32),
            pltpu.VMEM((C,), jnp.int32),
            pltpu.VMEM((B_d + 16,), jnp.int32),
            pltpu.VMEM((B_d + 16,), jnp.int32),
            pltpu.SemaphoreType.DMA,
            pltpu.SemaphoreType.DMA,
        ],
    )
    def k(h_hbm, src_hbm, olo_hbm, ohi_hbm, sum_hbm,
          acc, rbA, rbB, sxA, sxB, olo, ohi, smA, smB):
        cid = lax.axis_index("c")
        sid = lax.axis_index("s")
        wid = sid * 2 + cid
        zero16 = jnp.zeros((16,), jnp.float32)
        iota16 = lax.iota(jnp.int32, 16)

        def issue(base, sidx, rbuf, sem):
            pltpu.sync_copy(src_hbm.at[pl.ds(base, C)], sidx)
            pltpu.async_copy(h_hbm.at[sidx], rbuf, sem)

        def wait(sidx, rbuf, sem):
            pltpu.make_async_copy(h_hbm.at[sidx], rbuf, sem).wait()

        def process(rbuf, base):
            one16 = jnp.ones((16,), jnp.int32)
            zro16 = jnp.zeros((16,), jnp.int32)
            r0v = zro16
            r1v = zro16
            for g in range(B_d // 16):
                lo_g = olo[pl.ds(g * 16, 16)]
                hi_g = ohi[pl.ds(g * 16, 16)]
                r0v = r0v + jnp.where(hi_g <= base, one16, zro16)
                r1v = r1v + jnp.where(lo_g < base + C, one16, zro16)
            for sh in (8, 4, 2, 1):
                perm = jnp.bitwise_xor(iota16, sh)
                r0v = r0v + r0v.at[perm].get(mode="promise_in_bounds")
                r1v = r1v + r1v.at[perm].get(mode="promise_in_bounds")

            def rb(r, _):
                lo_r = olo[pl.ds(r, 16)][0]
                hi_r = ohi[pl.ds(r, 16)][0]
                es = jnp.maximum(lo_r - base, 0)
                ee = jnp.minimum(hi_r - base, C)

                @pl.when(ee > es)
                def _():
                    eee = jnp.maximum(ee, es)
                    for jh in range(JP):
                        wo = jh * WP
                        cols = [(wo + j2 * 16, d // 2 + wo + j2 * 16)
                                for j2 in range(WP // 16)]

                        def eb(e, carry):
                            out = []
                            for j2 in range(WP // 16):
                                w = rbuf[e, pl.ds(wo + j2 * 16, 16)]
                                lo = lax.bitcast_convert_type(
                                    jnp.left_shift(w, 16), jnp.float32)
                                hi = lax.bitcast_convert_type(
                                    w & jnp.int32(-65536), jnp.float32)
                                out.append(carry[2 * j2] + lo)
                                out.append(carry[2 * j2 + 1] + hi)
                            return tuple(out)

                        init = sum(
                            ((acc[r, pl.ds(ca, 16)], acc[r, pl.ds(cb, 16)])
                             for ca, cb in cols), ())
                        res = lax.fori_loop(es, eee, eb, init)
                        for j2, (ca, cb) in enumerate(cols):
                            acc[r, pl.ds(ca, 16)] = res[2 * j2]
                            acc[r, pl.ds(cb, 16)] = res[2 * j2 + 1]

                return 0

            lax.fori_loop(r0v[0], r1v[0], rb, 0)

        for kk in range(NBPW):
            b = wid * NBPW + kk
            lo = b * B_d
            pltpu.sync_copy(olo_hbm.at[pl.ds(lo, B_d)], olo.at[pl.ds(0, B_d)])
            pltpu.sync_copy(ohi_hbm.at[pl.ds(lo, B_d)], ohi.at[pl.ds(0, B_d)])

            def zrow(r, _):
                for j in range(d // 16):
                    acc[r, pl.ds(j * 16, 16)] = zero16
                return 0

            lax.fori_loop(0, B_d, zrow, 0)

            s0 = olo[pl.ds(0, 16)][0]
            e_end = ohi[pl.ds(B_d - 16, 16)][15]
            s0a = (s0 // 8) * 8
            nch = (e_end - s0a + C - 1) // C

            @pl.when(nch > 0)
            def _():
                issue(s0a, sxA, rbA, smA)

            def chunk(ct, _):
                b0 = s0a + ct * C

                @pl.when(ct % 2 == 0)
                def _():
                    @pl.when(ct + 1 < nch)
                    def _():
                        issue(b0 + C, sxB, rbB, smB)

                    wait(sxA, rbA, smA)
                    process(rbA, b0)

                @pl.when(ct % 2 == 1)
                def _():
                    @pl.when(ct + 1 < nch)
                    def _():
                        issue(b0 + C, sxA, rbA, smA)

                    wait(sxB, rbB, smB)
                    process(rbB, b0)

                return 0

            lax.fori_loop(0, nch, chunk, 0)
            pltpu.sync_copy(acc, sum_hbm.at[pl.ds(lo, B_d)])

    return k


def _dense_fused(n_dst, d_in, RB, d_res, with_proj, n_h):
    NBLK = n_dst // RB
    n_cb = _N_OUT // RB
    inv_n = 1.0 / float(n_dst)

    def body(h_ref, s_ref, ol_ref, oh_ref, ws_ref, wn_ref, g_ref, be_ref,
             r_ref, wr_ref, br_ref, o_ref, co_ref, ob_ref, y_scr, st_scr):
        p = pl.program_id(0)
        i = pl.program_id(1)

        @pl.when(p == 0)
        def _():
            cnt = (oh_ref[...] - ol_ref[...]).astype(jnp.float32)
            hn = s_ref[...] / jnp.maximum(cnt, 1.0)
            y = (jnp.dot(h_ref[...], ws_ref[...],
                         preferred_element_type=jnp.float32)
                 + jnp.dot(hn, wn_ref[...],
                           preferred_element_type=jnp.float32))
            y_scr[pl.ds(i * RB, RB), :] = y

            @pl.when(i == 0)
            def _():
                st_scr[...] = jnp.zeros_like(st_scr)

            st_scr[0:1, :] += jnp.sum(y, axis=0, keepdims=True)
            st_scr[1:2, :] += jnp.sum(y * y, axis=0, keepdims=True)

        @pl.when(p == 1)
        def _():
            mu = st_scr[0:1, :] * inv_n
            var = st_scr[1:2, :] * inv_n - mu * mu
            scale = g_ref[...] * lax.rsqrt(var + 1e-5)
            y = y_scr[pl.ds(i * RB, RB), :]
            hb = jnp.maximum((y - mu) * scale + be_ref[...], 0.0)

            @pl.when(i < n_cb)
            def _():
                co_ref[...] = hb

            if with_proj:
                res = (jnp.dot(r_ref[...], wr_ref[...],
                               preferred_element_type=jnp.float32)
                       + br_ref[...])
            else:
                res = r_ref[...]
            out = hb + res
            o_ref[...] = out
            a16 = out[:, :_NH // 2].astype(jnp.bfloat16)
            b16 = out[:, _NH // 2:].astype(jnp.bfloat16)
            aw = lax.bitcast_convert_type(a16, jnp.uint16).astype(jnp.int32)
            bw = lax.bitcast_convert_type(b16, jnp.uint16).astype(jnp.int32)
            ob_ref[...] = aw | (bw << 16)

    ph0 = lambda p, i: (jnp.where(p == 0, i, 0), 0)
    ph1 = lambda p, i: (jnp.where(p == 1, i, 0), 0)
    fix = lambda p, i: (0, 0)
    return pl.pallas_call(
        body,
        grid=(2, NBLK),
        in_specs=[
            pl.BlockSpec((RB, d_in), ph0),
            pl.BlockSpec((RB, d_in), ph0),
            pl.BlockSpec((RB, 1), ph0),
            pl.BlockSpec((RB, 1), ph0),
            pl.BlockSpec((d_in, _NH), fix),
            pl.BlockSpec((d_in, _NH), fix),
            pl.BlockSpec((1, _NH), fix),
            pl.BlockSpec((1, _NH), fix),
            pl.BlockSpec((RB, d_res), ph1),
            pl.BlockSpec((d_res, _NH), fix),
            pl.BlockSpec((1, _NH), fix),
        ],
        out_specs=[
            pl.BlockSpec((RB, _NH), ph1),
            pl.BlockSpec((RB, _NH),
                         lambda p, i: (jnp.where(p == 1, jnp.minimum(i, n_cb - 1), 0), 0)),
            pl.BlockSpec((RB, _NH // 2), ph1),
        ],
        out_shape=[
            jax.ShapeDtypeStruct((n_dst, _NH), jnp.float32),
            jax.ShapeDtypeStruct((_N_OUT, _NH), jnp.float32),
            jax.ShapeDtypeStruct((n_dst, _NH // 2), jnp.int32),
        ],
        scratch_shapes=[
            pltpu.VMEM((n_dst, _NH), jnp.float32),
            pltpu.VMEM((8, _NH), jnp.float32),
        ],
    )


def _mlp(d0, n_cls):
    d_cat = d0 + 3 * _NH

    def body(xp_ref, c1_ref, c2_ref, c3_ref, w1_ref, b1_ref, w2_ref, b2_ref,
             o_ref):
        h = (jnp.dot(xp_ref[...], w1_ref[0:d0, :],
                     preferred_element_type=jnp.float32)
             + jnp.dot(c1_ref[...], w1_ref[d0:d0 + _NH, :],
                       preferred_element_type=jnp.float32)
             + jnp.dot(c2_ref[...], w1_ref[d0 + _NH:d0 + 2 * _NH, :],
                       preferred_element_type=jnp.float32)
             + jnp.dot(c3_ref[...], w1_ref[d0 + 2 * _NH:d_cat, :],
                       preferred_element_type=jnp.float32)
             + b1_ref[...])
        o_ref[...] = (jnp.dot(h, w2_ref[...],
                              preferred_element_type=jnp.float32) + b2_ref[...])

    return pl.pallas_call(
        body,
        grid=(1,),
        in_specs=[
            pl.BlockSpec((_N_OUT, d0), lambda i: (0, 0)),
            pl.BlockSpec((_N_OUT, _NH), lambda i: (0, 0)),
            pl.BlockSpec((_N_OUT, _NH), lambda i: (0, 0)),
            pl.BlockSpec((_N_OUT, _NH), lambda i: (0, 0)),
            pl.BlockSpec((d_cat, 2 * n_cls), lambda i: (0, 0)),
            pl.BlockSpec((1, 2 * n_cls), lambda i: (0, 0)),
            pl.BlockSpec((2 * n_cls, n_cls), lambda i: (0, 0)),
            pl.BlockSpec((1, n_cls), lambda i: (0, 0)),
        ],
        out_specs=pl.BlockSpec((_N_OUT, n_cls), lambda i: (0, 0)),
        out_shape=jax.ShapeDtypeStruct((_N_OUT, n_cls), jnp.float32),
    )


def kernel(x, edge_src_0, edge_dst_0, edge_src_1, edge_dst_1, edge_src_2,
           edge_dst_2, W_self_0, W_neigh_0, gamma_0, beta_0, W_self_1,
           W_neigh_1, gamma_1, beta_1, W_self_2, W_neigh_2, gamma_2, beta_2,
           W_res, b_res, W_mlp1, b_mlp1, W_mlp2, b_mlp2):
    srcs = [edge_src_0, edge_src_1, edge_src_2]
    dsts = [edge_dst_0, edge_dst_1, edge_dst_2]
    Ws = [W_self_0, W_self_1, W_self_2]
    Wn = [W_neigh_0, W_neigh_1, W_neigh_2]
    gs = [gamma_0.reshape(1, -1), gamma_1.reshape(1, -1), gamma_2.reshape(1, -1)]
    bs = [beta_0.reshape(1, -1), beta_1.reshape(1, -1), beta_2.reshape(1, -1)]

    def _pack(hf):
        d2 = hf.shape[1] // 2
        h16 = hf.astype(jnp.bfloat16)
        aw = lax.bitcast_convert_type(h16[:, :d2], jnp.uint16).astype(jnp.int32)
        bw = lax.bitcast_convert_type(h16[:, d2:], jnp.uint16).astype(jnp.int32)
        return aw | (bw << 16)

    h = x
    h_bf = _pack(x)
    colls = [x[:_N_OUT]]
    for l in range(3):
        n_dst = _N_DST[l]
        d_in = h.shape[1]
        B_d = _BD[l]
        src, dst = srcs[l], dsts[l]
        E = src.shape[0]
        C = _CH[l]

        hist = jnp.zeros((n_dst,), jnp.int32).at[dst].add(1, mode="drop")
        off_hi = jnp.cumsum(hist, dtype=jnp.int32)
        off_lo = off_hi - hist
        src_p = jnp.concatenate([src, jnp.zeros((2 * C,), jnp.int32)])

        sums = _seg_sum_sc(d_in, n_dst, B_d, C, E)(h_bf, src_p, off_lo, off_hi)

        RB = 512
        if l == 0:
            h, coll, h_bf = _dense_fused(n_dst, d_in, RB, d_in, True, h.shape[0])(
                h, sums, off_lo.reshape(n_dst, 1),
                off_hi.reshape(n_dst, 1), Ws[l], Wn[l], gs[l], bs[l],
                h, W_res, b_res.reshape(1, -1))
        else:
            h, coll, h_bf = _dense_fused(n_dst, d_in, RB, _NH, False, h.shape[0])(
                h, sums, off_lo.reshape(n_dst, 1),
                off_hi.reshape(n_dst, 1), Ws[l], Wn[l], gs[l], bs[l],
                h, jnp.zeros((_NH, _NH), jnp.float32),
                jnp.zeros((1, _NH), jnp.float32))
        colls.append(coll)

    out = _mlp(x.shape[1], W_mlp2.shape[1])(
        colls[0], colls[1], colls[2], colls[3], W_mlp1,
        b_mlp1.reshape(1, -1), W_mlp2, b_mlp2.reshape(1, -1))
    return out
